# baseline (speedup 1.0000x reference)
import functools

import jax
import jax.numpy as jnp
from jax import lax
from jax.experimental import pallas as pl
from jax.experimental.pallas import tpu as pltpu
from jax.experimental.pallas import tpu_sc as plsc

_K = 4


@functools.lru_cache(maxsize=None)
def _make_kernel(n_lookups, d):
    info = plsc.get_sparse_core_info()
    nw = info.num_cores * info.num_subcores
    b_per_w = n_lookups // nw
    n_chunks = b_per_w // _K
    n_body = (n_chunks - 4) // 3
    assert n_chunks == 1 + 3 * n_body + 3

    mesh = plsc.VectorSubcoreMesh(core_axis_name="c", subcore_axis_name="s")

    @functools.partial(
        pl.kernel,
        mesh=mesh,
        out_type=jax.ShapeDtypeStruct((n_lookups, d), jnp.float32),
        scratch_types=[
            pltpu.VMEM((n_chunks, _K), jnp.int32),
            pltpu.VMEM((_K, d), jnp.float32),
            pltpu.VMEM((_K, d), jnp.float32),
            pltpu.VMEM((_K, d), jnp.float32),
            pltpu.SemaphoreType.DMA,
            pltpu.SemaphoreType.DMA,
            pltpu.SemaphoreType.DMA,
            pltpu.SemaphoreType.DMA,
            pltpu.SemaphoreType.DMA,
            pltpu.SemaphoreType.DMA,
        ],
    )
    def kern(idx_hbm, table_hbm, out_hbm, idx_v,
             buf_a, buf_b, buf_c, ga, gb, gc, sa, sb, sc):
        wid = lax.axis_index("s") * info.num_cores + lax.axis_index("c")
        base = wid * b_per_w
        pltpu.sync_copy(idx_hbm.at[wid], idx_v)

        bufs = (buf_a, buf_b, buf_c)
        gsems = (ga, gb, gc)
        ssems = (sa, sb, sc)

        def gather(c, t):
            pltpu.async_copy(
                table_hbm.at[idx_v.at[c]], bufs[t], gsems[t]
            )

        def gwait(t):
            pltpu.make_async_copy(
                table_hbm.at[pl.ds(0, _K)], bufs[t], gsems[t]
            ).wait()

        def scatter(c, t):
            pltpu.async_copy(
                bufs[t], out_hbm.at[pl.ds(base + c * _K, _K)], ssems[t]
            )

        def swait(t):
            pltpu.make_async_copy(
                bufs[t], out_hbm.at[pl.ds(0, _K)], ssems[t]
            ).wait()

        gather(0, 0)
        gather(1, 1)
        gwait(0)
        scatter(0, 0)
        gather(2, 2)

        def body(i, carry):
            c = 3 * i + 1
            for t in range(3):
                bt = (1 + t) % 3
                nxt = t % 3
                gwait(bt)
                scatter(c + t, bt)
                swait(nxt)
                gather(c + t + 2, nxt)
            return carry

        lax.fori_loop(0, n_body, body, 0)

        cl = n_chunks - 3
        gwait(1)
        scatter(cl, 1)
        swait(0)
        gather(cl + 2, 0)
        gwait(2)
        scatter(cl + 1, 2)
        gwait(0)
        scatter(cl + 2, 0)
        swait(1)
        swait(2)
        swait(0)

    return kern, nw, n_chunks


def kernel(indices, table):
    b, t = indices.shape
    n_lookups = b * t
    kern, nw, n_chunks = _make_kernel(n_lookups, table.shape[1])
    idx = indices.reshape(nw, n_chunks, _K).astype(jnp.int32)
    out = kern(idx, table)
    return out.reshape(b, t, table.shape[1])
